# Initial kernel scaffold; baseline (speedup 1.0000x reference)
import jax
import jax.numpy as jnp
from jax.experimental import pallas as pl


def kernel(x, expert_indices, W, b):
    raise NotImplementedError("write your pallas kernel here")


# speedup vs baseline: 2.2370x; 2.2370x over previous
import functools

import jax
import jax.numpy as jnp
from jax import lax
from jax.experimental import pallas as pl
from jax.experimental.pallas import tpu as pltpu
from jax.experimental.pallas import tpu_sc as plsc

B, T, D = 4, 2048, 2048
E, C = 8, 512
OUT = 16384
O_E = OUT // E
N_ROWS = B * E * C

NC, NS = 2, 16
NW = NC * NS
ROWS_PER_W = N_ROWS // NW
CHUNK = 32
N_CHUNKS = ROWS_PER_W // CHUNK


def _sc_gather(x2d, flat_idx):
    mesh = plsc.VectorSubcoreMesh(core_axis_name="c", subcore_axis_name="s")

    @functools.partial(
        pl.kernel,
        mesh=mesh,
        out_type=jax.ShapeDtypeStruct((N_ROWS, D), jnp.float32),
        scratch_types=[
            pltpu.VMEM((ROWS_PER_W,), jnp.int32),
            pltpu.VMEM((CHUNK, D), jnp.float32),
            pltpu.SemaphoreType.DMA,
        ],
    )
    def gather_kernel(x_hbm, idx_hbm, out_hbm, idx_v, rows_v, sem):
        wid = lax.axis_index("s") * NC + lax.axis_index("c")
        base = wid * ROWS_PER_W
        pltpu.sync_copy(idx_hbm.at[pl.ds(base, ROWS_PER_W)], idx_v)

        def body(c, carry):
            off = c * CHUNK
            cp = pltpu.make_async_copy(
                x_hbm.at[idx_v.at[pl.ds(off, CHUNK)]], rows_v, sem
            )
            cp.start()
            cp.wait()
            pltpu.sync_copy(rows_v, out_hbm.at[pl.ds(base + off, CHUNK)])
            return carry

        lax.fori_loop(0, N_CHUNKS, body, 0)

    return gather_kernel(x2d, flat_idx)


def _tc_matmul(g3, We, be):
    def mm_kernel(a_ref, w_ref, b_ref, o_ref):
        a = a_ref[0]
        w = w_ref[0]
        acc = lax.dot_general(
            a, w, (((1,), (1,)), ((), ())),
            preferred_element_type=jnp.float32,
        )
        o_ref[0, 0] = acc + b_ref[0]

    return pl.pallas_call(
        mm_kernel,
        grid=(E, B),
        in_specs=[
            pl.BlockSpec((1, C, D), lambda e, b: (e * B + b, 0, 0)),
            pl.BlockSpec((1, O_E, D), lambda e, b: (e, 0, 0)),
            pl.BlockSpec((1, 1, O_E), lambda e, b: (e, 0, 0)),
        ],
        out_specs=pl.BlockSpec((1, 1, C, O_E), lambda e, b: (b, e, 0, 0)),
        out_shape=jax.ShapeDtypeStruct((B, E, C, O_E), jnp.float32),
    )(g3, We, be)


def kernel(x, expert_indices, W, b):
    x2d = x.reshape(B * T, D)
    idx_ebc = jnp.transpose(expert_indices, (1, 0, 2))
    flat_idx = (
        idx_ebc + (jnp.arange(B, dtype=jnp.int32) * T)[None, :, None]
    ).reshape(N_ROWS)
    gathered = _sc_gather(x2d, flat_idx)
    g3 = gathered.reshape(E * B, C, D)
    return _tc_matmul(g3, W.reshape(E, O_E, D), b.reshape(E, 1, O_E))


# speedup vs baseline: 2.2876x; 1.0226x over previous
import functools

import jax
import jax.numpy as jnp
from jax import lax
from jax.experimental import pallas as pl
from jax.experimental.pallas import tpu as pltpu
from jax.experimental.pallas import tpu_sc as plsc

B, T, D = 4, 2048, 2048
E, C = 8, 512
OUT = 16384
O_E = OUT // E
N_ROWS = B * E * C

NC, NS = 2, 16
NW = NC * NS
ROWS_PER_W = N_ROWS // NW
CHUNK = 16
N_CHUNKS = ROWS_PER_W // CHUNK


def _sc_gather(x2d, flat_idx):
    mesh = plsc.VectorSubcoreMesh(core_axis_name="c", subcore_axis_name="s")

    @functools.partial(
        pl.kernel,
        mesh=mesh,
        out_type=jax.ShapeDtypeStruct((N_ROWS, D), jnp.float32),
        scratch_types=[
            pltpu.VMEM((ROWS_PER_W,), jnp.int32),
            pltpu.VMEM((CHUNK, D), jnp.float32),
            pltpu.VMEM((CHUNK, D), jnp.float32),
            pltpu.SemaphoreType.DMA,
            pltpu.SemaphoreType.DMA,
            pltpu.SemaphoreType.DMA,
            pltpu.SemaphoreType.DMA,
        ],
    )
    def gather_kernel(x_hbm, idx_hbm, out_hbm, idx_v, buf_a, buf_b, ga, gb, wa, wb):
        wid = lax.axis_index("s") * NC + lax.axis_index("c")
        base = wid * ROWS_PER_W
        pltpu.sync_copy(idx_hbm.at[pl.ds(base, ROWS_PER_W)], idx_v)

        bufs = (buf_a, buf_b)
        gsems = (ga, gb)
        wsems = (wa, wb)

        def gather_chunk(c):
            cp = pltpu.make_async_copy(
                x_hbm.at[idx_v.at[pl.ds(c * CHUNK, CHUNK)]], bufs[c % 2],
                gsems[c % 2],
            )
            cp.start()
            return cp

        def write_chunk(c):
            cp = pltpu.make_async_copy(
                bufs[c % 2], out_hbm.at[pl.ds(base + c * CHUNK, CHUNK)],
                wsems[c % 2],
            )
            cp.start()
            return cp

        g = [None] * N_CHUNKS
        w = [None] * N_CHUNKS
        g[0] = gather_chunk(0)
        g[1] = gather_chunk(1)
        g[0].wait()
        w[0] = write_chunk(0)
        for c in range(2, N_CHUNKS):
            w[c - 2].wait()
            g[c] = gather_chunk(c)
            g[c - 1].wait()
            w[c - 1] = write_chunk(c - 1)
        g[N_CHUNKS - 1].wait()
        w[N_CHUNKS - 1] = write_chunk(N_CHUNKS - 1)
        w[N_CHUNKS - 2].wait()
        w[N_CHUNKS - 1].wait()

    return gather_kernel(x2d, flat_idx)


def _tc_matmul(g3, We, be):
    def mm_kernel(a_ref, w_ref, b_ref, o_ref):
        a = a_ref[0]
        w = w_ref[0]
        acc = lax.dot_general(
            a, w, (((1,), (1,)), ((), ())),
            preferred_element_type=jnp.float32,
        )
        o_ref[0, 0] = acc + b_ref[0]

    return pl.pallas_call(
        mm_kernel,
        grid=(E, B),
        in_specs=[
            pl.BlockSpec((1, C, D), lambda e, b: (e * B + b, 0, 0)),
            pl.BlockSpec((1, O_E, D), lambda e, b: (e, 0, 0)),
            pl.BlockSpec((1, 1, O_E), lambda e, b: (e, 0, 0)),
        ],
        out_specs=pl.BlockSpec((1, 1, C, O_E), lambda e, b: (b, e, 0, 0)),
        out_shape=jax.ShapeDtypeStruct((B, E, C, O_E), jnp.float32),
    )(g3, We, be)


def kernel(x, expert_indices, W, b):
    x2d = x.reshape(B * T, D)
    idx_ebc = jnp.transpose(expert_indices, (1, 0, 2))
    flat_idx = (
        idx_ebc + (jnp.arange(B, dtype=jnp.int32) * T)[None, :, None]
    ).reshape(N_ROWS)
    gathered = _sc_gather(x2d, flat_idx)
    g3 = gathered.reshape(E * B, C, D)
    return _tc_matmul(g3, W.reshape(E, O_E, D), b.reshape(E, 1, O_E))
